# baseline (speedup 1.0000x reference)
import functools

import jax
import jax.numpy as jnp
from jax.experimental import pallas as pl
from jax.experimental.pallas import tpu as pltpu

N = 100000
D = 256
S = 1024
BLK = 16384
N_PAD = ((N + BLK - 1) // BLK) * BLK
NBLK = N_PAD // BLK
W = 256
ACC_ROWS = S + W + 8


def _fused_body(base8_ref, last_ref, idx_ref, x_ref,
                w1_ref, b1_ref, w2_ref, b2_ref,
                wr1_ref, br1_ref, wr2_ref, br2_ref, out_ref, acc_ref):
    i = pl.program_id(0)

    @pl.when(i == 0)
    def _init():
        acc_ref[...] = jnp.zeros_like(acc_ref)

    xb = x_ref[...].astype(jnp.bfloat16)
    h = jnp.maximum(
        jnp.dot(xb, w1_ref[...].astype(jnp.bfloat16),
                preferred_element_type=jnp.float32) + b1_ref[...], 0.0)
    xp = jnp.dot(h.astype(jnp.bfloat16), w2_ref[...].astype(jnp.bfloat16),
                 preferred_element_type=jnp.float32) + b2_ref[...]

    idx = idx_ref[0, 0, :]
    base8 = base8_ref[i]
    overflow = last_ref[i] - base8 >= W

    seg_iota = base8 + jax.lax.broadcasted_iota(jnp.int32, (W, BLK), 0)
    onehot = (seg_iota == idx[None, :]).astype(jnp.bfloat16)
    part = jnp.dot(onehot, xp.astype(jnp.bfloat16),
                   preferred_element_type=jnp.float32)
    off = pl.multiple_of(base8, 8)
    acc_ref[pl.ds(off, W), :] += part

    @pl.when(overflow)
    def _full():
        iota_s = jax.lax.broadcasted_iota(jnp.int32, (S, BLK), 0)
        beyond = idx >= base8 + W
        ohf = jnp.logical_and(iota_s == idx[None, :],
                              beyond[None, :]).astype(jnp.float32)
        acc_ref[pl.ds(0, S), :] += jnp.dot(
            ohf, xp, preferred_element_type=jnp.float32)

    @pl.when(i == NBLK - 1)
    def _rho():
        h2 = jnp.maximum(
            jnp.dot(acc_ref[pl.ds(0, S), :], wr1_ref[...],
                    preferred_element_type=jnp.float32) + br1_ref[...], 0.0)
        out_ref[...] = jnp.dot(h2, wr2_ref[...],
                               preferred_element_type=jnp.float32) + br2_ref[...]


@jax.jit
def _run(x, idx_i32, W_phi1, b_phi1, W_phi2, b_phi2,
         W_rho1, b_rho1, W_rho2, b_rho2):
    x_pad = jnp.pad(x, ((0, N_PAD - N), (0, 0)))
    idx_pad = jnp.pad(idx_i32, (0, N_PAD - N), constant_values=S)
    idx3 = idx_pad.reshape(NBLK, 1, BLK)
    base8 = (idx_pad[:: BLK] // 8) * 8
    last = idx_pad[BLK - 1:: BLK]

    wspec = pl.BlockSpec((D, D), lambda i: (0, 0))
    bspec = pl.BlockSpec((D,), lambda i: (0,))
    out = pl.pallas_call(
        _fused_body,
        grid=(NBLK,),
        in_specs=[
            pl.BlockSpec(memory_space=pltpu.SMEM),
            pl.BlockSpec(memory_space=pltpu.SMEM),
            pl.BlockSpec((1, 1, BLK), lambda i: (i, 0, 0)),
            pl.BlockSpec((BLK, D), lambda i: (i, 0)),
            wspec, bspec, wspec, bspec,
            wspec, bspec,
            pl.BlockSpec((D, D), lambda i: (0, 0)),
            pl.BlockSpec((D,), lambda i: (0,)),
        ],
        out_specs=pl.BlockSpec((S, D), lambda i: (0, 0)),
        out_shape=jax.ShapeDtypeStruct((S, D), jnp.float32),
        scratch_shapes=[pltpu.VMEM((ACC_ROWS, D), jnp.float32)],
    )(base8, last, idx3, x_pad, W_phi1, b_phi1, W_phi2, b_phi2,
      W_rho1, b_rho1, W_rho2, b_rho2)
    return out


def kernel(x, batch_idx, W_phi1, b_phi1, W_phi2, b_phi2,
           W_rho1, b_rho1, W_rho2, b_rho2):
    idx_i32 = batch_idx.astype(jnp.int32)
    return _run(x, idx_i32, W_phi1, b_phi1, W_phi2, b_phi2,
                W_rho1, b_rho1, W_rho2, b_rho2)


# speedup vs baseline: 1.1025x; 1.1025x over previous
import functools

import jax
import jax.numpy as jnp
from jax.experimental import pallas as pl
from jax.experimental.pallas import tpu as pltpu

N = 100000
D = 256
S = 1024
BLK = 8192
N_PAD = ((N + BLK - 1) // BLK) * BLK
NBLK = N_PAD // BLK
W = 128
ACC_ROWS = S + W + 8


def _fused_body(base8_ref, last_ref, idx_ref, x_ref,
                w1_ref, b1_ref, w2_ref, b2_ref,
                wr1_ref, br1_ref, wr2_ref, br2_ref, out_ref, acc_ref):
    i = pl.program_id(0)

    @pl.when(i == 0)
    def _init():
        acc_ref[...] = jnp.zeros_like(acc_ref)

    xb = x_ref[...].astype(jnp.bfloat16)
    h = jnp.maximum(
        jnp.dot(xb, w1_ref[...].astype(jnp.bfloat16),
                preferred_element_type=jnp.float32) + b1_ref[...], 0.0)
    xp = jnp.dot(h.astype(jnp.bfloat16), w2_ref[...].astype(jnp.bfloat16),
                 preferred_element_type=jnp.float32) + b2_ref[...]

    idx = idx_ref[0, 0, :]
    base8 = base8_ref[i]
    overflow = last_ref[i] - base8 >= W

    seg_iota = base8 + jax.lax.broadcasted_iota(jnp.int32, (W, BLK), 0)
    onehot = (seg_iota == idx[None, :]).astype(jnp.bfloat16)
    part = jnp.dot(onehot, xp.astype(jnp.bfloat16),
                   preferred_element_type=jnp.float32)
    off = pl.multiple_of(base8, 8)
    acc_ref[pl.ds(off, W), :] += part

    @pl.when(overflow)
    def _full():
        iota_s = jax.lax.broadcasted_iota(jnp.int32, (S, BLK), 0)
        beyond = idx >= base8 + W
        ohf = jnp.logical_and(iota_s == idx[None, :],
                              beyond[None, :]).astype(jnp.float32)
        acc_ref[pl.ds(0, S), :] += jnp.dot(
            ohf, xp, preferred_element_type=jnp.float32)

    @pl.when(i == NBLK - 1)
    def _rho():
        h2 = jnp.maximum(
            jnp.dot(acc_ref[pl.ds(0, S), :], wr1_ref[...],
                    preferred_element_type=jnp.float32) + br1_ref[...], 0.0)
        out_ref[...] = jnp.dot(h2, wr2_ref[...],
                               preferred_element_type=jnp.float32) + br2_ref[...]


@jax.jit
def _run(x, idx_i32, W_phi1, b_phi1, W_phi2, b_phi2,
         W_rho1, b_rho1, W_rho2, b_rho2):
    x_pad = jnp.pad(x, ((0, N_PAD - N), (0, 0)))
    idx_pad = jnp.pad(idx_i32, (0, N_PAD - N), constant_values=S)
    idx3 = idx_pad.reshape(NBLK, 1, BLK)
    base8 = (idx_pad[:: BLK] // 8) * 8
    last = idx_pad[BLK - 1:: BLK]

    wspec = pl.BlockSpec((D, D), lambda i: (0, 0))
    bspec = pl.BlockSpec((D,), lambda i: (0,))
    out = pl.pallas_call(
        _fused_body,
        grid=(NBLK,),
        in_specs=[
            pl.BlockSpec(memory_space=pltpu.SMEM),
            pl.BlockSpec(memory_space=pltpu.SMEM),
            pl.BlockSpec((1, 1, BLK), lambda i: (i, 0, 0)),
            pl.BlockSpec((BLK, D), lambda i: (i, 0)),
            wspec, bspec, wspec, bspec,
            wspec, bspec,
            pl.BlockSpec((D, D), lambda i: (0, 0)),
            pl.BlockSpec((D,), lambda i: (0,)),
        ],
        out_specs=pl.BlockSpec((S, D), lambda i: (0, 0)),
        out_shape=jax.ShapeDtypeStruct((S, D), jnp.float32),
        scratch_shapes=[pltpu.VMEM((ACC_ROWS, D), jnp.float32)],
    )(base8, last, idx3, x_pad, W_phi1, b_phi1, W_phi2, b_phi2,
      W_rho1, b_rho1, W_rho2, b_rho2)
    return out


def kernel(x, batch_idx, W_phi1, b_phi1, W_phi2, b_phi2,
           W_rho1, b_rho1, W_rho2, b_rho2):
    idx_i32 = batch_idx.astype(jnp.int32)
    return _run(x, idx_i32, W_phi1, b_phi1, W_phi2, b_phi2,
                W_rho1, b_rho1, W_rho2, b_rho2)
